# baseline (speedup 1.0000x reference)
import functools

import jax
import jax.numpy as jnp
from jax import lax
from jax.experimental import pallas as pl
from jax.experimental.pallas import tpu as pltpu
from jax.experimental.pallas import tpu_sc as plsc

BATCH = 16384
EMB_D = 64
NC = 2
NS = 16
NW = NC * NS
B_PER_W = BATCH // NW


def _gather_body(users_hbm, items_hbm, ut_hbm, it_hbm, ug_hbm, ig_hbm,
                 idx_u, idx_i, rows_u, rows_i, sem_u, sem_i):
    wid = lax.axis_index("s") * NC + lax.axis_index("c")
    base = wid * B_PER_W
    pltpu.sync_copy(users_hbm.at[pl.ds(base, B_PER_W)], idx_u)
    pltpu.sync_copy(items_hbm.at[pl.ds(base, B_PER_W)], idx_i)
    cu = pltpu.async_copy(ut_hbm.at[idx_u], rows_u, sem_u)
    ci = pltpu.async_copy(it_hbm.at[idx_i], rows_i, sem_i)
    cu.wait()
    ci.wait()
    pltpu.sync_copy(rows_u, ug_hbm.at[pl.ds(base, B_PER_W)])
    pltpu.sync_copy(rows_i, ig_hbm.at[pl.ds(base, B_PER_W)])


@functools.cache
def _sc_gather():
    return pl.kernel(
        _gather_body,
        out_type=(
            jax.ShapeDtypeStruct((BATCH, EMB_D), jnp.float32),
            jax.ShapeDtypeStruct((BATCH, EMB_D), jnp.float32),
        ),
        mesh=plsc.VectorSubcoreMesh(
            core_axis_name="c", subcore_axis_name="s",
            num_cores=NC, num_subcores=NS),
        scratch_types=[
            pltpu.VMEM((B_PER_W,), jnp.int32),
            pltpu.VMEM((B_PER_W,), jnp.int32),
            pltpu.VMEM((B_PER_W, EMB_D), jnp.float32),
            pltpu.VMEM((B_PER_W, EMB_D), jnp.float32),
            pltpu.SemaphoreType.DMA,
            pltpu.SemaphoreType.DMA,
        ],
        compiler_params=pltpu.CompilerParams(use_tc_tiling_on_sc=False),
    )


BLK = 2048


def _mlp_body(u_ref, i_ref, a_ref, w1u_ref, w1i_ref, w1a_ref, b1_ref,
              w2_ref, b2_ref, out_ref):
    h = jnp.dot(u_ref[...], w1u_ref[...], preferred_element_type=jnp.float32)
    h += jnp.dot(i_ref[...], w1i_ref[...], preferred_element_type=jnp.float32)
    h += jnp.dot(a_ref[...], w1a_ref[...], preferred_element_type=jnp.float32)
    h += b1_ref[...]
    h = jnp.maximum(h, 0.0)
    logits = jnp.sum(h * w2_ref[...], axis=1, keepdims=True) + b2_ref[...]
    out_ref[...] = jax.nn.sigmoid(logits)


def _mlp(u, i, a_pad, w1u, w1i, w1a, b1f, w2row, b2):
    n_blk = BATCH // BLK
    return pl.pallas_call(
        _mlp_body,
        grid=(n_blk,),
        in_specs=[
            pl.BlockSpec((BLK, EMB_D), lambda j: (j, 0)),
            pl.BlockSpec((BLK, EMB_D), lambda j: (j, 0)),
            pl.BlockSpec((BLK, 8), lambda j: (j, 0)),
            pl.BlockSpec((EMB_D, 128), lambda j: (0, 0)),
            pl.BlockSpec((EMB_D, 128), lambda j: (0, 0)),
            pl.BlockSpec((8, 128), lambda j: (0, 0)),
            pl.BlockSpec((1, 128), lambda j: (0, 0)),
            pl.BlockSpec((1, 128), lambda j: (0, 0)),
            pl.BlockSpec((1, 1), lambda j: (0, 0)),
        ],
        out_specs=pl.BlockSpec((BLK, 1), lambda j: (j, 0)),
        out_shape=jax.ShapeDtypeStruct((BATCH, 1), jnp.float32),
    )(u, i, a_pad, w1u, w1i, w1a, b1f, w2row, b2)


@jax.jit
def kernel(users, items, audio, user_table, item_table, Wa, ba, W1, b1, W2, b2):
    users = users.astype(jnp.int32)
    items = items.astype(jnp.int32)
    ug, ig = _sc_gather()(users, items, user_table, item_table)

    w1u = W1[:EMB_D]
    w1i = W1[EMB_D:2 * EMB_D]
    w1a4 = Wa @ W1[2 * EMB_D:]
    w1a = jnp.zeros((8, 128), jnp.float32).at[:4].set(w1a4)
    b1f = (b1 + ba @ W1[2 * EMB_D:]).reshape(1, 128)
    a_pad = jnp.zeros((BATCH, 8), jnp.float32).at[:, :4].set(audio)
    w2row = W2.reshape(1, 128)
    b2m = b2.reshape(1, 1)

    out = _mlp(ug, ig, a_pad, w1u, w1i, w1a, b1f, w2row, b2m)
    return out[:, 0]
